# Initial kernel scaffold; baseline (speedup 1.0000x reference)
import jax
import jax.numpy as jnp
from jax.experimental import pallas as pl


def kernel(q_len, k_len, rel_bias):
    raise NotImplementedError("write your pallas kernel here")


# speedup vs baseline: 76.0440x; 76.0440x over previous
import jax
import jax.numpy as jnp
from jax.experimental import pallas as pl
from jax.experimental.pallas import tpu as pltpu

_N_HEADS = 16
_MAX_DIST = 512
_Q = 2048
_K = 2048
_P_LEFT = _K - _MAX_DIST - 1
_E_LEN = 4096


def _expand_body(e8_ref, out_ref):
    for g in range(_Q // 8):
        s = (_K - 1) - 8 * g
        out_ref[0, 8 * g:8 * g + 8, :] = e8_ref[0, :, s:s + _K]


def kernel(q_len, k_len, rel_bias):
    d = (k_len - _K) - (q_len - _Q)
    u = jnp.arange(_E_LEN)[None, :]
    r = jnp.arange(8)[:, None]
    idx = jnp.clip(u - r - _P_LEFT + d, 0, 2 * _MAX_DIST)
    e8 = jnp.transpose(rel_bias[idx], (2, 0, 1))

    out = pl.pallas_call(
        _expand_body,
        grid=(_N_HEADS,),
        in_specs=[pl.BlockSpec((1, 8, _E_LEN), lambda h: (h, 0, 0))],
        out_specs=pl.BlockSpec((1, _Q, _K), lambda h: (h, 0, 0)),
        out_shape=jax.ShapeDtypeStruct((_N_HEADS, _Q, _K), rel_bias.dtype),
        compiler_params=pltpu.CompilerParams(
            dimension_semantics=("parallel",),
        ),
    )(e8)
    return out


# speedup vs baseline: 77.1899x; 1.0151x over previous
import jax
import jax.numpy as jnp
from jax.experimental import pallas as pl
from jax.experimental.pallas import tpu as pltpu

_N_HEADS = 16
_MAX_DIST = 512
_Q = 2048
_K = 2048
_P_LEFT = _K - _MAX_DIST - 1
_E8_LEN = 4224
_E128_LEN = 4096


def _expand_body(e8_ref, out_ref, e128_ref):
    for a in range(16):
        off = 127 - 8 * a
        e128_ref[8 * a:8 * a + 8, :] = e8_ref[0, :, off:off + _E128_LEN]
    for b in range(16):
        s = 1920 - 128 * b
        out_ref[0, 128 * b:128 * (b + 1), :] = e128_ref[:, s:s + _K]


def kernel(q_len, k_len, rel_bias):
    d = (k_len - _K) - (q_len - _Q)
    u = jnp.arange(_E8_LEN)[None, :]
    r = jnp.arange(8)[:, None]
    idx = jnp.clip(u - r - _P_LEFT + d, 0, 2 * _MAX_DIST)
    e8 = jnp.transpose(rel_bias[idx], (2, 0, 1))

    out = pl.pallas_call(
        _expand_body,
        grid=(_N_HEADS,),
        in_specs=[pl.BlockSpec((1, 8, _E8_LEN), lambda h: (h, 0, 0))],
        out_specs=pl.BlockSpec((1, _Q, _K), lambda h: (h, 0, 0)),
        out_shape=jax.ShapeDtypeStruct((_N_HEADS, _Q, _K), rel_bias.dtype),
        scratch_shapes=[pltpu.VMEM((128, _E128_LEN), rel_bias.dtype)],
        compiler_params=pltpu.CompilerParams(
            dimension_semantics=("parallel",),
        ),
    )(e8)
    return out


# speedup vs baseline: 167.7296x; 2.1729x over previous
import jax
import jax.numpy as jnp
from jax.experimental import pallas as pl
from jax.experimental.pallas import tpu as pltpu

_N_HEADS = 16
_MAX_DIST = 512
_Q = 2048
_K = 2048
_P_LEFT = _K - _MAX_DIST - 1
_E8_LEN = 4224
_E128_LEN = 4096


def _expand_body(e8_ref, out_ref, e128_ref):
    for a in range(16):
        off = 127 - 8 * a
        e128_ref[8 * a:8 * a + 8, :] = e8_ref[0, :, off:off + _E128_LEN]
    for b in range(16):
        s = 1920 - 128 * b
        out_ref[0, 128 * b:128 * (b + 1), :] = e128_ref[:, s:s + _K]


def kernel(q_len, k_len, rel_bias):
    d = (k_len - _K) - (q_len - _Q)
    vlen = _E8_LEN + 7
    w = jnp.arange(vlen)[:, None]
    mid_start = 7 + _P_LEFT - d
    base = jnp.where(w < mid_start, rel_bias[0][None, :], rel_bias[-1][None, :])
    v7 = jax.lax.dynamic_update_slice(base, rel_bias, (mid_start, 0))
    e8 = jnp.stack([jax.lax.slice(v7, (7 - r, 0), (7 - r + _E8_LEN, _N_HEADS))
                    for r in range(8)])
    e8 = jnp.transpose(e8, (2, 0, 1))

    out = pl.pallas_call(
        _expand_body,
        grid=(_N_HEADS,),
        in_specs=[pl.BlockSpec((1, 8, _E8_LEN), lambda h: (h, 0, 0))],
        out_specs=pl.BlockSpec((1, _Q, _K), lambda h: (h, 0, 0)),
        out_shape=jax.ShapeDtypeStruct((_N_HEADS, _Q, _K), rel_bias.dtype),
        scratch_shapes=[pltpu.VMEM((128, _E128_LEN), rel_bias.dtype)],
        compiler_params=pltpu.CompilerParams(
            dimension_semantics=("parallel",),
        ),
    )(e8)
    return out
